# Initial kernel scaffold; baseline (speedup 1.0000x reference)
import jax
import jax.numpy as jnp
from jax.experimental import pallas as pl


def kernel(queries, keys, values, attn_mask):
    raise NotImplementedError("write your pallas kernel here")


# speedup vs baseline: 3.1309x; 3.1309x over previous
import math

import jax
import jax.numpy as jnp
import numpy as np
from jax.experimental import pallas as pl
from jax.experimental.pallas import tpu as pltpu

_LQ = 2048
_LK = 2048
_FACTOR = 5
_U = _FACTOR * int(np.ceil(np.log(_LQ)))
_U_PART = _FACTOR * int(np.ceil(np.log(_LK)))
_UP = 64
_ROW_TILE = 256

_INDEX_SAMPLE = np.asarray(
    jax.random.randint(jax.random.key(42), (_LQ, _U_PART), 0, _LK)
)
_COUNTS = np.zeros((_LQ, _LK), np.int8)
np.add.at(_COUNTS, (np.arange(_LQ)[:, None], _INDEX_SAMPLE), 1)

_NEG_BIG = np.float32(-1e30)
_NEG_HUGE = np.float32(-3e38)


def _prob_attn_kernel(q_ref, k_ref, v_ref, c_ref, out_ref, m_scr, idx_scr):
    q = q_ref[0, :, :]
    k = k_ref[0, :, :]
    v = v_ref[0, :, :]

    nt = _LQ // _ROW_TILE
    for rb in range(nt):
        qt = q[rb * _ROW_TILE:(rb + 1) * _ROW_TILE, :]
        s_t = jax.lax.dot_general(
            qt, k, (((1,), (1,)), ((), ())),
            preferred_element_type=jnp.float32)
        c_t = c_ref[rb * _ROW_TILE:(rb + 1) * _ROW_TILE, :]
        c_f = c_t.astype(jnp.float32)
        masked = jnp.where(c_f > 0.0, s_t, _NEG_BIG)
        mx = jnp.max(masked, axis=-1, keepdims=True)
        sm = jnp.sum(s_t * c_f, axis=-1, keepdims=True)
        m_scr[rb * _ROW_TILE:(rb + 1) * _ROW_TILE, :] = mx - sm / _LK

    idx_scr[:, :] = jnp.full((_UP, 1), -1, jnp.int32)
    giota = jax.lax.broadcasted_iota(jnp.int32, (_LQ, 1), 0)

    def topk_body(i, carry):
        mv = m_scr[:, :]
        cur = jnp.max(mv, keepdims=True)
        cand = jnp.where(mv == cur, giota, jnp.int32(1 << 30))
        gidx = jnp.min(cand, keepdims=True)
        idx_scr[pl.ds(i, 1), :] = gidx
        m_scr[:, :] = jnp.where(giota == gidx, _NEG_HUGE, mv)
        return carry

    jax.lax.fori_loop(0, _U, topk_body, 0)

    idxv = idx_scr[:, :]
    cols = jax.lax.broadcasted_iota(jnp.int32, (_UP, _LK), 1)
    oh = (idxv == cols).astype(jnp.float32)
    q_red = jax.lax.dot_general(
        oh, q, (((1,), (0,)), ((), ())),
        preferred_element_type=jnp.float32)

    scores = jax.lax.dot_general(
        q_red, k, (((1,), (1,)), ((), ())),
        preferred_element_type=jnp.float32)
    scores = scores * np.float32(1.0 / math.sqrt(64))
    smax = jnp.max(scores, axis=-1, keepdims=True)
    e = jnp.exp(scores - smax)
    attn = e / jnp.sum(e, axis=-1, keepdims=True)
    update = jax.lax.dot_general(
        attn, v, (((1,), (0,)), ((), ())),
        preferred_element_type=jnp.float32)

    v_mean = jnp.sum(v, axis=0, keepdims=True) / _LK
    out_attn = jax.lax.dot_general(
        oh, update, (((0,), (0,)), ((), ())),
        preferred_element_type=jnp.float32)
    ones = jnp.ones((_UP, 1), jnp.float32)
    cov = jax.lax.dot_general(
        oh, ones, (((0,), (0,)), ((), ())),
        preferred_element_type=jnp.float32)
    out_ref[0, 0, :, :] = out_attn + (1.0 - cov) * v_mean


def kernel(queries, keys, values, attn_mask):
    del attn_mask
    B, Lq, H, D = queries.shape
    counts = jnp.asarray(_COUNTS)

    q_t = jnp.transpose(queries, (0, 2, 1, 3)).reshape(B * H, Lq, D)
    k_t = jnp.transpose(keys, (0, 2, 1, 3)).reshape(B * H, Lq, D)
    v_t = jnp.transpose(values, (0, 2, 1, 3)).reshape(B * H, Lq, D)

    grid = (B * H,)
    qkv_spec = pl.BlockSpec((1, Lq, D), lambda i: (i, 0, 0))
    c_spec = pl.BlockSpec((_LQ, _LK), lambda i: (0, 0))
    out_spec = pl.BlockSpec((1, 1, Lq, D), lambda i: (i // H, i % H, 0, 0))

    return pl.pallas_call(
        _prob_attn_kernel,
        grid=grid,
        in_specs=[qkv_spec, qkv_spec, qkv_spec, c_spec],
        out_specs=out_spec,
        out_shape=jax.ShapeDtypeStruct((B, H, Lq, D), jnp.float32),
        scratch_shapes=[
            pltpu.VMEM((_LQ, 1), jnp.float32),
            pltpu.VMEM((_UP, 1), jnp.int32),
        ],
    )(q_t, k_t, v_t, counts)


# speedup vs baseline: 5.8226x; 1.8597x over previous
import math

import jax
import jax.numpy as jnp
import numpy as np
from jax.experimental import pallas as pl
from jax.experimental.pallas import tpu as pltpu

_LQ = 2048
_LK = 2048
_FACTOR = 5
_U = _FACTOR * int(np.ceil(np.log(_LQ)))
_U_PART = _FACTOR * int(np.ceil(np.log(_LK)))
_UP = 64
_ROW_TILE = 256

_NEG_BIG = np.float32(-1e30)
_NEG_HUGE = np.float32(-3e38)


def _rotl32(x, d):
    return ((x << np.uint32(d)) | (x >> np.uint32(32 - d))).astype(np.uint32)


def _threefry2x32(k0, k1, x0, x1):
    x0 = x0.astype(np.uint32).copy()
    x1 = x1.astype(np.uint32).copy()
    ks = [np.uint32(k0), np.uint32(k1),
          np.uint32(np.uint32(k0) ^ np.uint32(k1) ^ np.uint32(0x1BD11BDA))]
    r1 = (13, 15, 26, 6)
    r2 = (17, 29, 16, 24)
    with np.errstate(over='ignore'):
        x0 += ks[0]
        x1 += ks[1]
        for i, rots in enumerate((r1, r2, r1, r2, r1)):
            for r in rots:
                x0 += x1
                x1 = _rotl32(x1, r)
                x1 ^= x0
            x0 += ks[(i + 1) % 3]
            x1 += ks[(i + 2) % 3] + np.uint32(i + 1)
    return x0, x1


def _np_randint_key42(shape, span):
    n = int(np.prod(shape))
    b1, b2 = _threefry2x32(np.uint32(0), np.uint32(42),
                           np.zeros(2, np.uint32),
                           np.arange(2, dtype=np.uint32))
    k2 = (b1[1], b2[1])
    o1, o2 = _threefry2x32(k2[0], k2[1],
                           np.zeros(n, np.uint32),
                           np.arange(n, dtype=np.uint32))
    return ((o1 ^ o2) % np.uint32(span)).astype(np.int32).reshape(shape)


_INDEX_SAMPLE = _np_randint_key42((_LQ, _U_PART), _LK)
_COUNTS = np.zeros((_LQ, _LK), np.float32)
np.add.at(_COUNTS, (np.arange(_LQ)[:, None], _INDEX_SAMPLE), 1.0)
_COUNTS_T = np.ascontiguousarray(_COUNTS.T)


def _prob_attn_kernel(q_ref, k_ref, v_ref, ct_ref, out_ref, m_scr, idx_scr):
    q = q_ref[0, :, :]
    k = k_ref[0, :, :]
    v = v_ref[0, :, :]

    nt = _LQ // _ROW_TILE
    for rb in range(nt):
        qt = q[rb * _ROW_TILE:(rb + 1) * _ROW_TILE, :]
        s_t = jax.lax.dot_general(
            k, qt, (((1,), (1,)), ((), ())),
            preferred_element_type=jnp.float32)
        c_t = ct_ref[:, rb * _ROW_TILE:(rb + 1) * _ROW_TILE]
        masked = jnp.where(c_t > 0.0, s_t, _NEG_BIG)
        mx = jnp.max(masked, axis=0, keepdims=True)
        sm = jnp.sum(s_t * c_t, axis=0, keepdims=True)
        m_scr[rb:rb + 1, :] = mx - sm / _LK

    idx_scr[:, :] = jnp.full((_UP, 1), -1, jnp.int32)
    giota = (jax.lax.broadcasted_iota(jnp.int32, (nt, _ROW_TILE), 0) * _ROW_TILE
             + jax.lax.broadcasted_iota(jnp.int32, (nt, _ROW_TILE), 1))

    def topk_body(i, carry):
        mv = m_scr[:, :]
        cur = jnp.max(mv, keepdims=True)
        cand = jnp.where(mv == cur, giota, jnp.int32(1 << 30))
        gidx = jnp.min(cand, keepdims=True)
        idx_scr[pl.ds(i, 1), :] = gidx
        m_scr[:, :] = jnp.where(giota == gidx, _NEG_HUGE, mv)
        return carry

    jax.lax.fori_loop(0, _U, topk_body, 0)

    idxv = idx_scr[:, :]
    cols = jax.lax.broadcasted_iota(jnp.int32, (_UP, _LK), 1)
    oh = (idxv == cols).astype(jnp.float32)
    q_red = jax.lax.dot_general(
        oh, q, (((1,), (0,)), ((), ())),
        preferred_element_type=jnp.float32)

    scores = jax.lax.dot_general(
        q_red, k, (((1,), (1,)), ((), ())),
        preferred_element_type=jnp.float32)
    scores = scores * np.float32(1.0 / math.sqrt(64))
    smax = jnp.max(scores, axis=-1, keepdims=True)
    e = jnp.exp(scores - smax)
    attn = e / jnp.sum(e, axis=-1, keepdims=True)
    update = jax.lax.dot_general(
        attn, v, (((1,), (0,)), ((), ())),
        preferred_element_type=jnp.float32)

    v_mean = jnp.sum(v, axis=0, keepdims=True) / _LK
    out_attn = jax.lax.dot_general(
        oh, update, (((0,), (0,)), ((), ())),
        preferred_element_type=jnp.float32)
    ones = jnp.ones((_UP, 1), jnp.float32)
    cov = jax.lax.dot_general(
        oh, ones, (((0,), (0,)), ((), ())),
        preferred_element_type=jnp.float32)
    out_ref[0, 0, :, :] = out_attn + (1.0 - cov) * v_mean


def kernel(queries, keys, values, attn_mask):
    del attn_mask
    B, Lq, H, D = queries.shape
    counts_t = jnp.asarray(_COUNTS_T)

    q_t = jnp.transpose(queries, (0, 2, 1, 3)).reshape(B * H, Lq, D)
    k_t = jnp.transpose(keys, (0, 2, 1, 3)).reshape(B * H, Lq, D)
    v_t = jnp.transpose(values, (0, 2, 1, 3)).reshape(B * H, Lq, D)

    grid = (B * H,)
    qkv_spec = pl.BlockSpec((1, Lq, D), lambda i: (i, 0, 0))
    c_spec = pl.BlockSpec((_LK, _LQ), lambda i: (0, 0))
    out_spec = pl.BlockSpec((1, 1, Lq, D), lambda i: (i // H, i % H, 0, 0))

    return pl.pallas_call(
        _prob_attn_kernel,
        grid=grid,
        in_specs=[qkv_spec, qkv_spec, qkv_spec, c_spec],
        out_specs=out_spec,
        out_shape=jax.ShapeDtypeStruct((B, H, Lq, D), jnp.float32),
        scratch_shapes=[
            pltpu.VMEM((_LQ // _ROW_TILE, _ROW_TILE), jnp.float32),
            pltpu.VMEM((_UP, 1), jnp.int32),
        ],
    )(q_t, k_t, v_t, counts_t)
